# baseline (speedup 1.0000x reference)
import functools

import jax
import jax.numpy as jnp
from jax import lax
from jax.experimental import pallas as pl
from jax.experimental.pallas import tpu as pltpu
from jax.experimental.pallas import tpu_sc as plsc

N_NODES = 10000
N_PAD = 10240
N_EDGES = 320000
IN_DIM = 128
MEM_DIM = 32
HEADS = 2
OUT_C = 32
HO = HEADS * OUT_C
ACC_W = 128
SCALE = 1.0 / (32.0 ** 0.5)

NUM_CORES = 2
NUM_SUBCORES = 16
NW = NUM_CORES * NUM_SUBCORES
EPW = N_EDGES // NW
CHUNK = 80
NCHUNK = EPW // CHUNK
ROWS_PER_TILE = N_PAD // NUM_SUBCORES


def _node_tables_body(x_ref, mem_ref, wq_ref, wk_ref, wv_ref, wm_ref,
                      dst_ref, src_ref):
    x = x_ref[...]
    mm = jnp.dot(mem_ref[...], wm_ref[...], preferred_element_type=jnp.float32)
    xq = jnp.dot(x, wq_ref[...], preferred_element_type=jnp.float32)
    dst_ref[...] = jnp.concatenate([xq, jnp.zeros_like(xq)], axis=1)
    kv = jnp.dot(x, wk_ref[...], preferred_element_type=jnp.float32) + mm
    vv = jnp.dot(x, wv_ref[...], preferred_element_type=jnp.float32) + mm
    src_ref[...] = jnp.concatenate([kv, vv], axis=1)


def _node_tables(x, memory, Wq, Wk, Wv, Wm):
    return pl.pallas_call(
        _node_tables_body,
        out_shape=(
            jax.ShapeDtypeStruct((N_NODES, 2 * HO), jnp.float32),
            jax.ShapeDtypeStruct((N_NODES, 2 * HO), jnp.float32),
        ),
    )(x, memory, Wq, Wk, Wv, Wm)


E_ROWS = 2560
E_BR = 256


def _etab_body(t_ref, w_ref, b_ref, wt_ref, out_ref):
    tb = t_ref[...]
    te = jnp.cos(tb[:, :, None] * w_ref[0, :][None, None, :]
                 + b_ref[0, :][None, None, :])
    out_ref[...] = lax.dot_general(
        te, wt_ref[...], (((2,), (0,)), ((), ())),
        preferred_element_type=jnp.float32)


def _etab(t2, w2, b2, Wt):
    grid = (E_ROWS // E_BR,)
    return pl.pallas_call(
        _etab_body,
        grid=grid,
        in_specs=[
            pl.BlockSpec((E_BR, 128), lambda i: (i, 0)),
            pl.BlockSpec((1, MEM_DIM), lambda i: (0, 0)),
            pl.BlockSpec((1, MEM_DIM), lambda i: (0, 0)),
            pl.BlockSpec((MEM_DIM, HO), lambda i: (0, 0)),
        ],
        out_specs=pl.BlockSpec((E_BR, 128, HO), lambda i: (i, 0, 0)),
        out_shape=jax.ShapeDtypeStruct((E_ROWS, 128, HO), jnp.float32),
    )(t2, w2, b2, Wt)


def _sc_edges(src, dst, etab, srctab, dsttab, zeros):
    mesh = plsc.VectorSubcoreMesh(core_axis_name="c", subcore_axis_name="s")

    @functools.partial(
        pl.kernel,
        mesh=mesh,
        out_type=jax.ShapeDtypeStruct((NUM_CORES, N_PAD, ACC_W), jnp.float32),
        scratch_types=[
            pltpu.VMEM_SHARED((N_PAD, ACC_W), jnp.float32),
            pltpu.VMEM((CHUNK,), jnp.int32),
            pltpu.VMEM((CHUNK,), jnp.int32),
            pltpu.VMEM((CHUNK, 2 * HO), jnp.float32),
            pltpu.VMEM((CHUNK, 2 * HO), jnp.float32),
            pltpu.VMEM((CHUNK, HO), jnp.float32),
            pltpu.VMEM((CHUNK, ACC_W), jnp.float32),
            pltpu.SemaphoreType.DMA,
            pltpu.SemaphoreType.DMA,
        ],
    )
    def k(src_hbm, dst_hbm, etab_hbm, srctab_hbm, dsttab_hbm, zeros_hbm,
          out_hbm, acc_sh, sidx_v, didx_v, srow_v, drow_v, erow_v, crow_v,
          sem_a, sem_b):
        ci = lax.axis_index("c")
        si = lax.axis_index("s")
        wid = si * NUM_CORES + ci

        row0 = si * ROWS_PER_TILE
        pltpu.sync_copy(zeros_hbm.at[pl.ds(row0, ROWS_PER_TILE)],
                        acc_sh.at[pl.ds(row0, ROWS_PER_TILE)])
        plsc.subcore_barrier()

        lanes = lax.iota(jnp.int32, 16)
        perms = [lanes ^ d for d in (1, 2, 4, 8)]

        dnums = lax.GatherDimensionNumbers(
            offset_dims=(), collapsed_slice_dims=(0,), start_index_map=(0,))

        def lane_gather(v, p):
            return lax.gather(v, p[:, None], dnums, (1,),
                              mode=lax.GatherScatterMode.PROMISE_IN_BOUNDS)

        def allsum(v):
            for p in perms:
                v = v + lane_gather(v, p)
            return v

        def chunk_body(j, _):
            base = wid * EPW + j * CHUNK
            pltpu.sync_copy(src_hbm.at[pl.ds(base, CHUNK)], sidx_v)
            pltpu.sync_copy(dst_hbm.at[pl.ds(base, CHUNK)], didx_v)
            ga = pltpu.async_copy(srctab_hbm.at[sidx_v], srow_v, sem_a)
            gb = pltpu.async_copy(dsttab_hbm.at[didx_v], drow_v, sem_b)
            pltpu.sync_copy(etab_hbm.at[pl.ds(base, CHUNK)], erow_v)
            ga.wait()
            gb.wait()

            def edge_body(i, _):
                e0 = erow_v[i, pl.ds(0, 16)]
                e1 = erow_v[i, pl.ds(16, 16)]
                e2 = erow_v[i, pl.ds(32, 16)]
                e3 = erow_v[i, pl.ds(48, 16)]
                q0 = drow_v[i, pl.ds(0, 16)]
                q1 = drow_v[i, pl.ds(16, 16)]
                q2 = drow_v[i, pl.ds(32, 16)]
                q3 = drow_v[i, pl.ds(48, 16)]
                k0 = srow_v[i, pl.ds(0, 16)] + e0
                k1 = srow_v[i, pl.ds(16, 16)] + e1
                k2 = srow_v[i, pl.ds(32, 16)] + e2
                k3 = srow_v[i, pl.ds(48, 16)] + e3
                w0 = jnp.exp(allsum(q0 * k0 + q1 * k1) * SCALE)
                w1 = jnp.exp(allsum(q2 * k2 + q3 * k3) * SCALE)
                v0 = srow_v[i, pl.ds(64, 16)] + e0
                v1 = srow_v[i, pl.ds(80, 16)] + e1
                v2 = srow_v[i, pl.ds(96, 16)] + e2
                v3 = srow_v[i, pl.ds(112, 16)] + e3
                crow_v[i, pl.ds(0, 16)] = w0 * v0
                crow_v[i, pl.ds(16, 16)] = w0 * v1
                crow_v[i, pl.ds(32, 16)] = w1 * v2
                crow_v[i, pl.ds(48, 16)] = w1 * v3
                crow_v[i, pl.ds(64, 16)] = jnp.where(
                    lanes == 0, w0, jnp.where(lanes == 1, w1, 0.0))
                return 0

            lax.fori_loop(0, CHUNK, edge_body, 0)
            pltpu.sync_copy(crow_v, acc_sh.at[didx_v], add=True)
            return 0

        lax.fori_loop(0, NCHUNK, chunk_body, 0)
        plsc.subcore_barrier()
        pltpu.sync_copy(acc_sh.at[pl.ds(row0, ROWS_PER_TILE)],
                        out_hbm.at[ci, pl.ds(row0, ROWS_PER_TILE)])

    return k(src, dst, etab, srctab, dsttab, zeros)


def _final_body(p_ref, out_ref):
    p = p_ref[...]
    acc = p[0] + p[1]
    num = acc[:, 0:HO]
    d0 = acc[:, HO:HO + 1]
    d1 = acc[:, HO + 1:HO + 2]
    out_ref[...] = jnp.concatenate(
        [num[:, 0:OUT_C] / (d0 + 1e-16), num[:, OUT_C:HO] / (d1 + 1e-16)],
        axis=1)


def _finalize(parts):
    return pl.pallas_call(
        _final_body,
        out_shape=jax.ShapeDtypeStruct((N_PAD, HO), jnp.float32),
    )(parts)


def kernel(src, dst, t, x, memory, w_time, b_time, Wq, Wk, Wv, Wt, Wm):
    src = src.astype(jnp.int32)
    dst = dst.astype(jnp.int32)
    dsttab, srctab = _node_tables(x, memory, Wq, Wk, Wv, Wm)
    t_pad = jnp.concatenate(
        [t, jnp.zeros((E_ROWS * 128 - N_EDGES,), jnp.float32)]
    ).reshape(E_ROWS, 128)
    etab = _etab(t_pad, w_time.reshape(1, MEM_DIM),
                 b_time.reshape(1, MEM_DIM), Wt).reshape(E_ROWS * 128, HO)
    zeros = jnp.zeros((N_PAD, ACC_W), jnp.float32)
    parts = _sc_edges(src, dst, etab, srctab, dsttab, zeros)
    return _finalize(parts)[:N_NODES]


# speedup vs baseline: 1.7570x; 1.7570x over previous
import functools

import jax
import jax.numpy as jnp
from jax import lax
from jax.experimental import pallas as pl
from jax.experimental.pallas import tpu as pltpu
from jax.experimental.pallas import tpu_sc as plsc

N_NODES = 10000
N_PAD = 10240
N_EDGES = 320000
IN_DIM = 128
MEM_DIM = 32
HEADS = 2
OUT_C = 32
HO = HEADS * OUT_C
ACC_W = 128
SCALE = 1.0 / (32.0 ** 0.5)

NUM_CORES = 2
NUM_SUBCORES = 16
NW = NUM_CORES * NUM_SUBCORES
EPW = N_EDGES // NW
CHUNK = 80
NCHUNK = EPW // CHUNK
ROWS_PER_TILE = N_PAD // NUM_SUBCORES


def _node_tables_body(x_ref, mem_ref, wq_ref, wk_ref, wv_ref, wm_ref,
                      dst_ref, src_ref):
    x = x_ref[...]
    mm = jnp.dot(mem_ref[...], wm_ref[...], preferred_element_type=jnp.float32)
    xq = jnp.dot(x, wq_ref[...], preferred_element_type=jnp.float32)
    dst_ref[...] = jnp.concatenate([xq, jnp.zeros_like(xq)], axis=1)
    kv = jnp.dot(x, wk_ref[...], preferred_element_type=jnp.float32) + mm
    vv = jnp.dot(x, wv_ref[...], preferred_element_type=jnp.float32) + mm
    src_ref[...] = jnp.concatenate([kv, vv], axis=1)


def _node_tables(x, memory, Wq, Wk, Wv, Wm):
    return pl.pallas_call(
        _node_tables_body,
        out_shape=(
            jax.ShapeDtypeStruct((N_NODES, 2 * HO), jnp.float32),
            jax.ShapeDtypeStruct((N_NODES, 2 * HO), jnp.float32),
        ),
    )(x, memory, Wq, Wk, Wv, Wm)


E_ROWS = 2560
E_BR = 256

import numpy as _np

_rg = _np.linspace(-3.3, 3.3, 20001)
_COS_C = _np.linalg.lstsq(
    _rg[:, None] ** (2 * _np.arange(9)[None, :]), _np.cos(_rg), rcond=None
)[0].astype(_np.float32)
_TWO_PI_HI = _np.float32(6.28125)
_TWO_PI_LO = _np.float32(2 * _np.pi - 6.28125)
_INV_2PI = _np.float32(1.0 / (2 * _np.pi))


def _fast_cos(x):
    n = lax.round(x * _INV_2PI, lax.RoundingMethod.TO_NEAREST_EVEN)
    r = (x - n * _TWO_PI_HI) - n * _TWO_PI_LO
    r2 = r * r
    acc = jnp.full_like(r2, _COS_C[8])
    for k in range(7, -1, -1):
        acc = acc * r2 + _COS_C[k]
    return acc


def _etab_body(t_ref, w_ref, b_ref, wt_ref, out_ref):
    tb = t_ref[...]
    te = _fast_cos(tb[:, :, None] * w_ref[0, :][None, None, :]
                   + b_ref[0, :][None, None, :])
    out_ref[...] = lax.dot_general(
        te, wt_ref[...], (((2,), (0,)), ((), ())),
        preferred_element_type=jnp.float32)


def _etab(t2, w2, b2, Wt):
    grid = (E_ROWS // E_BR,)
    return pl.pallas_call(
        _etab_body,
        grid=grid,
        in_specs=[
            pl.BlockSpec((E_BR, 128), lambda i: (i, 0)),
            pl.BlockSpec((1, MEM_DIM), lambda i: (0, 0)),
            pl.BlockSpec((1, MEM_DIM), lambda i: (0, 0)),
            pl.BlockSpec((MEM_DIM, HO), lambda i: (0, 0)),
        ],
        out_specs=pl.BlockSpec((E_BR, 128, HO), lambda i: (i, 0, 0)),
        out_shape=jax.ShapeDtypeStruct((E_ROWS, 128, HO), jnp.float32),
    )(t2, w2, b2, Wt)


def _sc_edges(src, dst, etab, srctab, dsttab, zeros):
    mesh = plsc.VectorSubcoreMesh(core_axis_name="c", subcore_axis_name="s")

    @functools.partial(
        pl.kernel,
        mesh=mesh,
        out_type=jax.ShapeDtypeStruct((NUM_CORES, N_PAD, ACC_W), jnp.float32),
        scratch_types=[
            pltpu.VMEM_SHARED((N_PAD, ACC_W), jnp.float32),
            pltpu.VMEM((CHUNK,), jnp.int32),
            pltpu.VMEM((CHUNK,), jnp.int32),
            pltpu.VMEM((CHUNK, 2 * HO), jnp.float32),
            pltpu.VMEM((CHUNK, 2 * HO), jnp.float32),
            pltpu.VMEM((CHUNK, HO), jnp.float32),
            pltpu.VMEM((CHUNK, ACC_W), jnp.float32),
            pltpu.SemaphoreType.DMA,
            pltpu.SemaphoreType.DMA,
        ],
    )
    def k(src_hbm, dst_hbm, etab_hbm, srctab_hbm, dsttab_hbm, zeros_hbm,
          out_hbm, acc_sh, sidx_v, didx_v, srow_v, drow_v, erow_v, crow_v,
          sem_a, sem_b):
        ci = lax.axis_index("c")
        si = lax.axis_index("s")
        wid = si * NUM_CORES + ci

        row0 = si * ROWS_PER_TILE
        pltpu.sync_copy(zeros_hbm.at[pl.ds(row0, ROWS_PER_TILE)],
                        acc_sh.at[pl.ds(row0, ROWS_PER_TILE)])
        plsc.subcore_barrier()

        lanes = lax.iota(jnp.int32, 16)
        perms = [lanes ^ d for d in (1, 2, 4, 8)]

        dnums = lax.GatherDimensionNumbers(
            offset_dims=(), collapsed_slice_dims=(0,), start_index_map=(0,))

        def lane_gather(v, p):
            return lax.gather(v, p[:, None], dnums, (1,),
                              mode=lax.GatherScatterMode.PROMISE_IN_BOUNDS)

        def allsum(v):
            for p in perms:
                v = v + lane_gather(v, p)
            return v

        def chunk_body(j, _):
            base = wid * EPW + j * CHUNK
            pltpu.sync_copy(src_hbm.at[pl.ds(base, CHUNK)], sidx_v)
            pltpu.sync_copy(dst_hbm.at[pl.ds(base, CHUNK)], didx_v)
            ga = pltpu.async_copy(srctab_hbm.at[sidx_v], srow_v, sem_a)
            gb = pltpu.async_copy(dsttab_hbm.at[didx_v], drow_v, sem_b)
            pltpu.sync_copy(etab_hbm.at[pl.ds(base, CHUNK)], erow_v)
            ga.wait()
            gb.wait()

            def edge_body(i, _):
                e0 = erow_v[i, pl.ds(0, 16)]
                e1 = erow_v[i, pl.ds(16, 16)]
                e2 = erow_v[i, pl.ds(32, 16)]
                e3 = erow_v[i, pl.ds(48, 16)]
                q0 = drow_v[i, pl.ds(0, 16)]
                q1 = drow_v[i, pl.ds(16, 16)]
                q2 = drow_v[i, pl.ds(32, 16)]
                q3 = drow_v[i, pl.ds(48, 16)]
                k0 = srow_v[i, pl.ds(0, 16)] + e0
                k1 = srow_v[i, pl.ds(16, 16)] + e1
                k2 = srow_v[i, pl.ds(32, 16)] + e2
                k3 = srow_v[i, pl.ds(48, 16)] + e3
                w0 = jnp.exp(allsum(q0 * k0 + q1 * k1) * SCALE)
                w1 = jnp.exp(allsum(q2 * k2 + q3 * k3) * SCALE)
                v0 = srow_v[i, pl.ds(64, 16)] + e0
                v1 = srow_v[i, pl.ds(80, 16)] + e1
                v2 = srow_v[i, pl.ds(96, 16)] + e2
                v3 = srow_v[i, pl.ds(112, 16)] + e3
                crow_v[i, pl.ds(0, 16)] = w0 * v0
                crow_v[i, pl.ds(16, 16)] = w0 * v1
                crow_v[i, pl.ds(32, 16)] = w1 * v2
                crow_v[i, pl.ds(48, 16)] = w1 * v3
                crow_v[i, pl.ds(64, 16)] = jnp.where(
                    lanes == 0, w0, jnp.where(lanes == 1, w1, 0.0))
                return 0

            lax.fori_loop(0, CHUNK, edge_body, 0)
            pltpu.sync_copy(crow_v, acc_sh.at[didx_v], add=True)
            return 0

        lax.fori_loop(0, NCHUNK, chunk_body, 0)
        plsc.subcore_barrier()
        pltpu.sync_copy(acc_sh.at[pl.ds(row0, ROWS_PER_TILE)],
                        out_hbm.at[ci, pl.ds(row0, ROWS_PER_TILE)])

    return k(src, dst, etab, srctab, dsttab, zeros)


def _final_body(p_ref, out_ref):
    p = p_ref[...]
    acc = p[0] + p[1]
    num = acc[:, 0:HO]
    d0 = acc[:, HO:HO + 1]
    d1 = acc[:, HO + 1:HO + 2]
    out_ref[...] = jnp.concatenate(
        [num[:, 0:OUT_C] / (d0 + 1e-16), num[:, OUT_C:HO] / (d1 + 1e-16)],
        axis=1)


def _finalize(parts):
    return pl.pallas_call(
        _final_body,
        out_shape=jax.ShapeDtypeStruct((N_PAD, HO), jnp.float32),
    )(parts)


def kernel(src, dst, t, x, memory, w_time, b_time, Wq, Wk, Wv, Wt, Wm):
    src = src.astype(jnp.int32)
    dst = dst.astype(jnp.int32)
    dsttab, srctab = _node_tables(x, memory, Wq, Wk, Wv, Wm)
    t_pad = jnp.concatenate(
        [t, jnp.zeros((E_ROWS * 128 - N_EDGES,), jnp.float32)]
    ).reshape(E_ROWS, 128)
    etab = _etab(t_pad, w_time.reshape(1, MEM_DIM),
                 b_time.reshape(1, MEM_DIM), Wt).reshape(E_ROWS * 128, HO)
    zeros = jnp.zeros((N_PAD, ACC_W), jnp.float32)
    parts = _sc_edges(src, dst, etab, srctab, dsttab, zeros)
    return _finalize(parts)[:N_NODES]


# speedup vs baseline: 2.5716x; 1.4636x over previous
import functools

import jax
import jax.numpy as jnp
from jax import lax
from jax.experimental import pallas as pl
from jax.experimental.pallas import tpu as pltpu
from jax.experimental.pallas import tpu_sc as plsc

N_NODES = 10000
N_PAD = 10240
N_EDGES = 320000
IN_DIM = 128
MEM_DIM = 32
HEADS = 2
OUT_C = 32
HO = HEADS * OUT_C
ACC_W = 128
SCALE = 1.0 / (32.0 ** 0.5)

NUM_CORES = 2
NUM_SUBCORES = 16
NW = NUM_CORES * NUM_SUBCORES
EPW = N_EDGES // NW
CHUNK = 80
NCHUNK = EPW // CHUNK
ROWS_PER_TILE = N_PAD // NUM_SUBCORES


def _node_tables_body(x_ref, mem_ref, wq_ref, wk_ref, wv_ref, wm_ref,
                      dst_ref, src_ref):
    x = x_ref[...]
    mm = jnp.dot(mem_ref[...], wm_ref[...], preferred_element_type=jnp.float32)
    xq = jnp.dot(x, wq_ref[...], preferred_element_type=jnp.float32)
    dst_ref[...] = jnp.concatenate([xq, jnp.zeros_like(xq)], axis=1)
    kv = jnp.dot(x, wk_ref[...], preferred_element_type=jnp.float32) + mm
    vv = jnp.dot(x, wv_ref[...], preferred_element_type=jnp.float32) + mm
    src_ref[...] = jnp.concatenate([kv, vv], axis=1)


def _node_tables(x, memory, Wq, Wk, Wv, Wm):
    return pl.pallas_call(
        _node_tables_body,
        out_shape=(
            jax.ShapeDtypeStruct((N_NODES, 2 * HO), jnp.float32),
            jax.ShapeDtypeStruct((N_NODES, 2 * HO), jnp.float32),
        ),
    )(x, memory, Wq, Wk, Wv, Wm)


E_ROWS = 2560
E_BR = 256

import numpy as _np

_rg = _np.linspace(-3.3, 3.3, 20001)
_COS_C = _np.linalg.lstsq(
    _rg[:, None] ** (2 * _np.arange(9)[None, :]), _np.cos(_rg), rcond=None
)[0].astype(_np.float32)
_TWO_PI_HI = _np.float32(6.28125)
_TWO_PI_LO = _np.float32(2 * _np.pi - 6.28125)
_INV_2PI = _np.float32(1.0 / (2 * _np.pi))


def _fast_cos(x):
    n = lax.round(x * _INV_2PI, lax.RoundingMethod.TO_NEAREST_EVEN)
    r = (x - n * _TWO_PI_HI) - n * _TWO_PI_LO
    r2 = r * r
    acc = jnp.full_like(r2, _COS_C[8])
    for k in range(7, -1, -1):
        acc = acc * r2 + _COS_C[k]
    return acc


def _etab_body(t_ref, w_ref, b_ref, wt_ref, out_ref):
    tb = t_ref[...]
    te = _fast_cos(tb[:, :, None] * w_ref[0, :][None, None, :]
                   + b_ref[0, :][None, None, :])
    out_ref[...] = lax.dot_general(
        te, wt_ref[...], (((2,), (0,)), ((), ())),
        preferred_element_type=jnp.float32)


def _etab(t2, w2, b2, Wt):
    grid = (E_ROWS // E_BR,)
    return pl.pallas_call(
        _etab_body,
        grid=grid,
        in_specs=[
            pl.BlockSpec((E_BR, 128), lambda i: (i, 0)),
            pl.BlockSpec((1, MEM_DIM), lambda i: (0, 0)),
            pl.BlockSpec((1, MEM_DIM), lambda i: (0, 0)),
            pl.BlockSpec((MEM_DIM, HO), lambda i: (0, 0)),
        ],
        out_specs=pl.BlockSpec((E_BR, 128, HO), lambda i: (i, 0, 0)),
        out_shape=jax.ShapeDtypeStruct((E_ROWS, 128, HO), jnp.float32),
    )(t2, w2, b2, Wt)


def _sc_edges(src, dst, etab, srctab, dsttab, zeros):
    mesh = plsc.VectorSubcoreMesh(core_axis_name="c", subcore_axis_name="s")

    H_CH = CHUNK // 2

    @functools.partial(
        pl.kernel,
        mesh=mesh,
        out_type=jax.ShapeDtypeStruct((NUM_CORES, N_PAD, ACC_W), jnp.float32),
        scratch_types=[
            pltpu.VMEM_SHARED((N_PAD, ACC_W), jnp.float32),
            pltpu.VMEM((2, 2, H_CH), jnp.int32),
            pltpu.VMEM((2, 2, H_CH), jnp.int32),
            pltpu.VMEM((CHUNK, 2 * HO), jnp.float32),
            pltpu.VMEM((CHUNK, 2 * HO), jnp.float32),
            pltpu.VMEM((CHUNK, HO), jnp.float32),
            pltpu.VMEM((CHUNK, ACC_W), jnp.float32),
            pltpu.SemaphoreType.DMA,
            pltpu.SemaphoreType.DMA,
            pltpu.SemaphoreType.DMA,
            pltpu.SemaphoreType.DMA,
            pltpu.SemaphoreType.DMA,
        ],
    )
    def k(src_hbm, dst_hbm, etab_hbm, srctab_hbm, dsttab_hbm, zeros_hbm,
          out_hbm, acc_sh, sidx_v, didx_v, srow_v, drow_v, erow_v, crow_v,
          sem_a, sem_b, sem_i, sem_sa, sem_sb):
        ci = lax.axis_index("c")
        si = lax.axis_index("s")
        wid = si * NUM_CORES + ci

        row0 = si * ROWS_PER_TILE
        pltpu.sync_copy(zeros_hbm.at[pl.ds(row0, ROWS_PER_TILE)],
                        acc_sh.at[pl.ds(row0, ROWS_PER_TILE)])
        plsc.subcore_barrier()

        lanes = lax.iota(jnp.int32, 16)
        perms = [lanes ^ d for d in (1, 2, 4, 8)]

        dnums = lax.GatherDimensionNumbers(
            offset_dims=(), collapsed_slice_dims=(0,), start_index_map=(0,))

        def lane_gather(v, p):
            return lax.gather(v, p[:, None], dnums, (1,),
                              mode=lax.GatherScatterMode.PROMISE_IN_BOUNDS)

        def allsum(v):
            for p in perms:
                v = v + lane_gather(v, p)
            return v

        def idx_copies(j, par):
            base = wid * EPW + j * CHUNK
            return [
                pltpu.make_async_copy(src_hbm.at[pl.ds(base, H_CH)],
                                      sidx_v.at[par, 0], sem_i),
                pltpu.make_async_copy(src_hbm.at[pl.ds(base + H_CH, H_CH)],
                                      sidx_v.at[par, 1], sem_i),
                pltpu.make_async_copy(dst_hbm.at[pl.ds(base, H_CH)],
                                      didx_v.at[par, 0], sem_i),
                pltpu.make_async_copy(dst_hbm.at[pl.ds(base + H_CH, H_CH)],
                                      didx_v.at[par, 1], sem_i),
            ]

        def half_gathers(j, par, h, sem):
            base = wid * EPW + j * CHUNK + h * H_CH
            off = h * H_CH
            return [
                pltpu.make_async_copy(srctab_hbm.at[sidx_v.at[par, h]],
                                      srow_v.at[pl.ds(off, H_CH)], sem),
                pltpu.make_async_copy(dsttab_hbm.at[didx_v.at[par, h]],
                                      drow_v.at[pl.ds(off, H_CH)], sem),
                pltpu.make_async_copy(etab_hbm.at[pl.ds(base, H_CH)],
                                      erow_v.at[pl.ds(off, H_CH)], sem),
            ]

        def half_scatter(par, h, sem):
            off = h * H_CH
            return pltpu.make_async_copy(crow_v.at[pl.ds(off, H_CH)],
                                         acc_sh.at[didx_v.at[par, h]], sem)

        def compute_half(h):
            off = h * H_CH

            def edge_body(i, _):
                r = off + i
                e0 = erow_v[r, pl.ds(0, 16)]
                e1 = erow_v[r, pl.ds(16, 16)]
                e2 = erow_v[r, pl.ds(32, 16)]
                e3 = erow_v[r, pl.ds(48, 16)]
                q0 = drow_v[r, pl.ds(0, 16)]
                q1 = drow_v[r, pl.ds(16, 16)]
                q2 = drow_v[r, pl.ds(32, 16)]
                q3 = drow_v[r, pl.ds(48, 16)]
                k0 = srow_v[r, pl.ds(0, 16)] + e0
                k1 = srow_v[r, pl.ds(16, 16)] + e1
                k2 = srow_v[r, pl.ds(32, 16)] + e2
                k3 = srow_v[r, pl.ds(48, 16)] + e3
                w0 = jnp.exp(allsum(q0 * k0 + q1 * k1) * SCALE)
                w1 = jnp.exp(allsum(q2 * k2 + q3 * k3) * SCALE)
                v0 = srow_v[r, pl.ds(64, 16)] + e0
                v1 = srow_v[r, pl.ds(80, 16)] + e1
                v2 = srow_v[r, pl.ds(96, 16)] + e2
                v3 = srow_v[r, pl.ds(112, 16)] + e3
                crow_v[r, pl.ds(0, 16)] = w0 * v0
                crow_v[r, pl.ds(16, 16)] = w0 * v1
                crow_v[r, pl.ds(32, 16)] = w1 * v2
                crow_v[r, pl.ds(48, 16)] = w1 * v3
                crow_v[r, pl.ds(64, 16)] = jnp.where(
                    lanes == 0, w0, jnp.where(lanes == 1, w1, 0.0))
                return 0

            lax.fori_loop(0, H_CH, edge_body, 0)

        def chunk_steps(j, par, first, last):
            if not first:
                half_scatter(1 - par, 0, sem_sa).wait()
                half_scatter(1 - par, 1, sem_sb).wait()
            if not last:
                for c in idx_copies(j + 1, 1 - par):
                    c.start()
            for c in half_gathers(j, par, 0, sem_a):
                c.wait()
            for c in half_gathers(j, par, 1, sem_b):
                c.start()
            compute_half(0)
            half_scatter(par, 0, sem_sa).start(add=True)
            if not last:
                for c in idx_copies(j + 1, 1 - par):
                    c.wait()
                for c in half_gathers(j + 1, 1 - par, 0, sem_a):
                    c.start()
            for c in half_gathers(j, par, 1, sem_b):
                c.wait()
            compute_half(1)
            half_scatter(par, 1, sem_sb).start(add=True)

        for c in idx_copies(0, 0):
            c.start()
        for c in idx_copies(0, 0):
            c.wait()
        for c in half_gathers(0, 0, 0, sem_a):
            c.start()

        def pair_body(jj, _):
            j = 2 * jj
            chunk_steps(j, 0, first=False, last=False)
            chunk_steps(j + 1, 1, first=False, last=False)
            return 0

        chunk_steps(0, 0, first=True, last=False)
        chunk_steps(1, 1, first=False, last=False)

        lax.fori_loop(1, (NCHUNK - 1) // 2, pair_body, 0)

        chunk_steps(NCHUNK - 1, 0, first=False, last=True)
        half_scatter(0, 0, sem_sa).wait()
        half_scatter(0, 1, sem_sb).wait()

        plsc.subcore_barrier()
        pltpu.sync_copy(acc_sh.at[pl.ds(row0, ROWS_PER_TILE)],
                        out_hbm.at[ci, pl.ds(row0, ROWS_PER_TILE)])

    return k(src, dst, etab, srctab, dsttab, zeros)


def _final_body(p_ref, out_ref):
    p = p_ref[...]
    acc = p[0] + p[1]
    num = acc[:, 0:HO]
    d0 = acc[:, HO:HO + 1]
    d1 = acc[:, HO + 1:HO + 2]
    out_ref[...] = jnp.concatenate(
        [num[:, 0:OUT_C] / (d0 + 1e-16), num[:, OUT_C:HO] / (d1 + 1e-16)],
        axis=1)


def _finalize(parts):
    return pl.pallas_call(
        _final_body,
        out_shape=jax.ShapeDtypeStruct((N_PAD, HO), jnp.float32),
    )(parts)


def kernel(src, dst, t, x, memory, w_time, b_time, Wq, Wk, Wv, Wt, Wm):
    src = src.astype(jnp.int32)
    dst = dst.astype(jnp.int32)
    dsttab, srctab = _node_tables(x, memory, Wq, Wk, Wv, Wm)
    t_pad = jnp.concatenate(
        [t, jnp.zeros((E_ROWS * 128 - N_EDGES,), jnp.float32)]
    ).reshape(E_ROWS, 128)
    etab = _etab(t_pad, w_time.reshape(1, MEM_DIM),
                 b_time.reshape(1, MEM_DIM), Wt).reshape(E_ROWS * 128, HO)
    zeros = jnp.zeros((N_PAD, ACC_W), jnp.float32)
    parts = _sc_edges(src, dst, etab, srctab, dsttab, zeros)
    return _finalize(parts)[:N_NODES]


# speedup vs baseline: 2.5763x; 1.0019x over previous
import functools

import jax
import jax.numpy as jnp
from jax import lax
from jax.experimental import pallas as pl
from jax.experimental.pallas import tpu as pltpu
from jax.experimental.pallas import tpu_sc as plsc

N_NODES = 10000
N_PAD = 10240
N_EDGES = 320000
IN_DIM = 128
MEM_DIM = 32
HEADS = 2
OUT_C = 32
HO = HEADS * OUT_C
ACC_W = 128
SCALE = 1.0 / (32.0 ** 0.5)

NUM_CORES = 2
NUM_SUBCORES = 16
NW = NUM_CORES * NUM_SUBCORES
EPW = N_EDGES // NW
CHUNK = 80
NCHUNK = EPW // CHUNK
ROWS_PER_TILE = N_PAD // NUM_SUBCORES


def _node_tables_body(x_ref, mem_ref, wq_ref, wk_ref, wv_ref, wm_ref,
                      dst_ref, src_ref):
    x = x_ref[...]
    mm = jnp.dot(mem_ref[...], wm_ref[...], preferred_element_type=jnp.float32)
    xq = jnp.dot(x, wq_ref[...], preferred_element_type=jnp.float32)
    dst_ref[...] = jnp.concatenate([xq, jnp.zeros_like(xq)], axis=1)
    kv = jnp.dot(x, wk_ref[...], preferred_element_type=jnp.float32) + mm
    vv = jnp.dot(x, wv_ref[...], preferred_element_type=jnp.float32) + mm
    src_ref[...] = jnp.concatenate([kv, vv], axis=1)


def _node_tables(x, memory, Wq, Wk, Wv, Wm):
    return pl.pallas_call(
        _node_tables_body,
        out_shape=(
            jax.ShapeDtypeStruct((N_NODES, 2 * HO), jnp.float32),
            jax.ShapeDtypeStruct((N_NODES, 2 * HO), jnp.float32),
        ),
    )(x, memory, Wq, Wk, Wv, Wm)


E_ROWS = 2560
E_BR = 256

import numpy as _np

_rg = _np.linspace(-3.3, 3.3, 20001)
_COS_C = _np.linalg.lstsq(
    _rg[:, None] ** (2 * _np.arange(9)[None, :]), _np.cos(_rg), rcond=None
)[0].astype(_np.float32)
_TWO_PI_HI = _np.float32(6.28125)
_TWO_PI_LO = _np.float32(2 * _np.pi - 6.28125)
_INV_2PI = _np.float32(1.0 / (2 * _np.pi))


def _fast_cos(x):
    n = lax.round(x * _INV_2PI, lax.RoundingMethod.TO_NEAREST_EVEN)
    r = (x - n * _TWO_PI_HI) - n * _TWO_PI_LO
    r2 = r * r
    acc = jnp.full_like(r2, _COS_C[8])
    for k in range(7, -1, -1):
        acc = acc * r2 + _COS_C[k]
    return acc


def _etab_body(t_ref, w_ref, b_ref, wt_ref, out_ref):
    tb = t_ref[...]
    te = _fast_cos(tb[:, :, None] * w_ref[0, :][None, None, :]
                   + b_ref[0, :][None, None, :])
    out_ref[...] = lax.dot_general(
        te, wt_ref[...], (((2,), (0,)), ((), ())),
        preferred_element_type=jnp.float32)


def _etab(t2, w2, b2, Wt):
    grid = (E_ROWS // E_BR,)
    return pl.pallas_call(
        _etab_body,
        grid=grid,
        in_specs=[
            pl.BlockSpec((E_BR, 128), lambda i: (i, 0)),
            pl.BlockSpec((1, MEM_DIM), lambda i: (0, 0)),
            pl.BlockSpec((1, MEM_DIM), lambda i: (0, 0)),
            pl.BlockSpec((MEM_DIM, HO), lambda i: (0, 0)),
        ],
        out_specs=pl.BlockSpec((E_BR, 128, HO), lambda i: (i, 0, 0)),
        out_shape=jax.ShapeDtypeStruct((E_ROWS, 128, HO), jnp.float32),
    )(t2, w2, b2, Wt)


def _sc_edges(src, dst, etab, srctab, dsttab, zeros):
    mesh = plsc.VectorSubcoreMesh(core_axis_name="c", subcore_axis_name="s")

    H_CH = CHUNK // 2

    @functools.partial(
        pl.kernel,
        mesh=mesh,
        out_type=jax.ShapeDtypeStruct((NUM_CORES, N_PAD, ACC_W), jnp.float32),
        scratch_types=[
            pltpu.VMEM_SHARED((N_PAD, ACC_W), jnp.float32),
            pltpu.VMEM((2, 2, H_CH), jnp.int32),
            pltpu.VMEM((2, 2, H_CH), jnp.int32),
            pltpu.VMEM((CHUNK, 2 * HO), jnp.float32),
            pltpu.VMEM((CHUNK, 2 * HO), jnp.float32),
            pltpu.VMEM((CHUNK, HO), jnp.float32),
            pltpu.VMEM((CHUNK, ACC_W), jnp.float32),
            pltpu.SemaphoreType.DMA,
            pltpu.SemaphoreType.DMA,
            pltpu.SemaphoreType.DMA,
            pltpu.SemaphoreType.DMA,
            pltpu.SemaphoreType.DMA,
        ],
    )
    def k(src_hbm, dst_hbm, etab_hbm, srctab_hbm, dsttab_hbm, zeros_hbm,
          out_hbm, acc_sh, sidx_v, didx_v, srow_v, drow_v, erow_v, crow_v,
          sem_a, sem_b, sem_i, sem_sa, sem_sb):
        ci = lax.axis_index("c")
        si = lax.axis_index("s")
        wid = si * NUM_CORES + ci

        row0 = si * ROWS_PER_TILE
        pltpu.sync_copy(zeros_hbm.at[pl.ds(row0, ROWS_PER_TILE)],
                        acc_sh.at[pl.ds(row0, ROWS_PER_TILE)])
        plsc.subcore_barrier()

        lanes = lax.iota(jnp.int32, 16)
        perms = [lanes ^ d for d in (1, 2, 4, 8)]

        dnums = lax.GatherDimensionNumbers(
            offset_dims=(), collapsed_slice_dims=(0,), start_index_map=(0,))

        def lane_gather(v, p):
            return lax.gather(v, p[:, None], dnums, (1,),
                              mode=lax.GatherScatterMode.PROMISE_IN_BOUNDS)

        def allsum(v):
            for p in perms:
                v = v + lane_gather(v, p)
            return v

        def idx_copies(j, par):
            base = wid * EPW + j * CHUNK
            return [
                pltpu.make_async_copy(src_hbm.at[pl.ds(base, H_CH)],
                                      sidx_v.at[par, 0], sem_i),
                pltpu.make_async_copy(src_hbm.at[pl.ds(base + H_CH, H_CH)],
                                      sidx_v.at[par, 1], sem_i),
                pltpu.make_async_copy(dst_hbm.at[pl.ds(base, H_CH)],
                                      didx_v.at[par, 0], sem_i),
                pltpu.make_async_copy(dst_hbm.at[pl.ds(base + H_CH, H_CH)],
                                      didx_v.at[par, 1], sem_i),
            ]

        def half_gathers(j, par, h, sem):
            base = wid * EPW + j * CHUNK + h * H_CH
            off = h * H_CH
            return [
                pltpu.make_async_copy(srctab_hbm.at[sidx_v.at[par, h]],
                                      srow_v.at[pl.ds(off, H_CH)], sem),
                pltpu.make_async_copy(dsttab_hbm.at[didx_v.at[par, h]],
                                      drow_v.at[pl.ds(off, H_CH)], sem),
                pltpu.make_async_copy(etab_hbm.at[pl.ds(base, H_CH)],
                                      erow_v.at[pl.ds(off, H_CH)], sem),
            ]

        def half_scatter(par, h, sem):
            off = h * H_CH
            return pltpu.make_async_copy(crow_v.at[pl.ds(off, H_CH)],
                                         acc_sh.at[didx_v.at[par, h]], sem)

        def compute_half(h):
            off = h * H_CH

            @plsc.parallel_loop(0, H_CH, step=1, unroll=2)
            def edge_body(i):
                r = off + i
                e0 = erow_v[r, pl.ds(0, 16)]
                e1 = erow_v[r, pl.ds(16, 16)]
                e2 = erow_v[r, pl.ds(32, 16)]
                e3 = erow_v[r, pl.ds(48, 16)]
                q0 = drow_v[r, pl.ds(0, 16)]
                q1 = drow_v[r, pl.ds(16, 16)]
                q2 = drow_v[r, pl.ds(32, 16)]
                q3 = drow_v[r, pl.ds(48, 16)]
                k0 = srow_v[r, pl.ds(0, 16)] + e0
                k1 = srow_v[r, pl.ds(16, 16)] + e1
                k2 = srow_v[r, pl.ds(32, 16)] + e2
                k3 = srow_v[r, pl.ds(48, 16)] + e3
                w0 = jnp.exp(allsum(q0 * k0 + q1 * k1) * SCALE)
                w1 = jnp.exp(allsum(q2 * k2 + q3 * k3) * SCALE)
                v0 = srow_v[r, pl.ds(64, 16)] + e0
                v1 = srow_v[r, pl.ds(80, 16)] + e1
                v2 = srow_v[r, pl.ds(96, 16)] + e2
                v3 = srow_v[r, pl.ds(112, 16)] + e3
                crow_v[r, pl.ds(0, 16)] = w0 * v0
                crow_v[r, pl.ds(16, 16)] = w0 * v1
                crow_v[r, pl.ds(32, 16)] = w1 * v2
                crow_v[r, pl.ds(48, 16)] = w1 * v3
                crow_v[r, pl.ds(64, 16)] = jnp.where(
                    lanes == 0, w0, jnp.where(lanes == 1, w1, 0.0))

        def chunk_steps(j, par, first, last):
            if not first:
                half_scatter(1 - par, 0, sem_sa).wait()
                half_scatter(1 - par, 1, sem_sb).wait()
            if not last:
                for c in idx_copies(j + 1, 1 - par):
                    c.start()
            for c in half_gathers(j, par, 0, sem_a):
                c.wait()
            for c in half_gathers(j, par, 1, sem_b):
                c.start()
            compute_half(0)
            half_scatter(par, 0, sem_sa).start(add=True)
            if not last:
                for c in idx_copies(j + 1, 1 - par):
                    c.wait()
                for c in half_gathers(j + 1, 1 - par, 0, sem_a):
                    c.start()
            for c in half_gathers(j, par, 1, sem_b):
                c.wait()
            compute_half(1)
            half_scatter(par, 1, sem_sb).start(add=True)

        for c in idx_copies(0, 0):
            c.start()
        for c in idx_copies(0, 0):
            c.wait()
        for c in half_gathers(0, 0, 0, sem_a):
            c.start()

        def pair_body(jj, _):
            j = 2 * jj
            chunk_steps(j, 0, first=False, last=False)
            chunk_steps(j + 1, 1, first=False, last=False)
            return 0

        chunk_steps(0, 0, first=True, last=False)
        chunk_steps(1, 1, first=False, last=False)

        lax.fori_loop(1, (NCHUNK - 1) // 2, pair_body, 0)

        chunk_steps(NCHUNK - 1, 0, first=False, last=True)
        half_scatter(0, 0, sem_sa).wait()
        half_scatter(0, 1, sem_sb).wait()

        plsc.subcore_barrier()
        pltpu.sync_copy(acc_sh.at[pl.ds(row0, ROWS_PER_TILE)],
                        out_hbm.at[ci, pl.ds(row0, ROWS_PER_TILE)])

    return k(src, dst, etab, srctab, dsttab, zeros)


def _final_body(p_ref, out_ref):
    p = p_ref[...]
    acc = p[0] + p[1]
    num = acc[:, 0:HO]
    d0 = acc[:, HO:HO + 1]
    d1 = acc[:, HO + 1:HO + 2]
    out_ref[...] = jnp.concatenate(
        [num[:, 0:OUT_C] / (d0 + 1e-16), num[:, OUT_C:HO] / (d1 + 1e-16)],
        axis=1)


def _finalize(parts):
    return pl.pallas_call(
        _final_body,
        out_shape=jax.ShapeDtypeStruct((N_PAD, HO), jnp.float32),
    )(parts)


def kernel(src, dst, t, x, memory, w_time, b_time, Wq, Wk, Wv, Wt, Wm):
    src = src.astype(jnp.int32)
    dst = dst.astype(jnp.int32)
    dsttab, srctab = _node_tables(x, memory, Wq, Wk, Wv, Wm)
    t_pad = jnp.concatenate(
        [t, jnp.zeros((E_ROWS * 128 - N_EDGES,), jnp.float32)]
    ).reshape(E_ROWS, 128)
    etab = _etab(t_pad, w_time.reshape(1, MEM_DIM),
                 b_time.reshape(1, MEM_DIM), Wt).reshape(E_ROWS * 128, HO)
    zeros = jnp.zeros((N_PAD, ACC_W), jnp.float32)
    parts = _sc_edges(src, dst, etab, srctab, dsttab, zeros)
    return _finalize(parts)[:N_NODES]


# speedup vs baseline: 2.6911x; 1.0445x over previous
import functools

import jax
import jax.numpy as jnp
from jax import lax
from jax.experimental import pallas as pl
from jax.experimental.pallas import tpu as pltpu
from jax.experimental.pallas import tpu_sc as plsc

N_NODES = 10000
N_PAD = 10240
N_EDGES = 320000
IN_DIM = 128
MEM_DIM = 32
HEADS = 2
OUT_C = 32
HO = HEADS * OUT_C
ACC_W = 128
SCALE = 1.0 / (32.0 ** 0.5)

NUM_CORES = 2
NUM_SUBCORES = 16
NW = NUM_CORES * NUM_SUBCORES
EPW = N_EDGES // NW
CHUNK = 80
NCHUNK = EPW // CHUNK
ROWS_PER_TILE = N_PAD // NUM_SUBCORES


def _node_tables_body(x_ref, mem_ref, wq_ref, wk_ref, wv_ref, wm_ref,
                      dst_ref, src_ref):
    x = x_ref[...]
    mm = jnp.dot(mem_ref[...], wm_ref[...], preferred_element_type=jnp.float32)
    xq = jnp.dot(x, wq_ref[...], preferred_element_type=jnp.float32)
    dst_ref[...] = jnp.concatenate([xq, jnp.zeros_like(xq)], axis=1)
    kv = jnp.dot(x, wk_ref[...], preferred_element_type=jnp.float32) + mm
    vv = jnp.dot(x, wv_ref[...], preferred_element_type=jnp.float32) + mm
    src_ref[...] = jnp.concatenate([kv, vv], axis=1)


def _node_tables(x, memory, Wq, Wk, Wv, Wm):
    return pl.pallas_call(
        _node_tables_body,
        out_shape=(
            jax.ShapeDtypeStruct((N_NODES, 2 * HO), jnp.float32),
            jax.ShapeDtypeStruct((N_NODES, 2 * HO), jnp.float32),
        ),
    )(x, memory, Wq, Wk, Wv, Wm)


E_ROWS = 2560
E_BR = 256

import numpy as _np

_rg = _np.linspace(-3.3, 3.3, 20001)
_COS_C = _np.linalg.lstsq(
    _rg[:, None] ** (2 * _np.arange(7)[None, :]), _np.cos(_rg), rcond=None
)[0].astype(_np.float32)
_TWO_PI_HI = _np.float32(6.28125)
_TWO_PI_LO = _np.float32(2 * _np.pi - 6.28125)
_INV_2PI = _np.float32(1.0 / (2 * _np.pi))


def _fast_cos(x):
    n = lax.round(x * _INV_2PI, lax.RoundingMethod.TO_NEAREST_EVEN)
    r = (x - n * _TWO_PI_HI) - n * _TWO_PI_LO
    r2 = r * r
    acc = jnp.full_like(r2, _COS_C[6])
    for k in range(5, -1, -1):
        acc = acc * r2 + _COS_C[k]
    return acc


def _etab_body(t_ref, w_ref, b_ref, wt_ref, out_ref):
    tb = t_ref[...]
    te = _fast_cos(tb[:, :, None] * w_ref[0, :][None, None, :]
                   + b_ref[0, :][None, None, :])
    out_ref[...] = lax.dot_general(
        te, wt_ref[...], (((2,), (0,)), ((), ())),
        preferred_element_type=jnp.float32)


def _etab(t2, w2, b2, Wt):
    grid = (E_ROWS // E_BR,)
    return pl.pallas_call(
        _etab_body,
        grid=grid,
        in_specs=[
            pl.BlockSpec((E_BR, 128), lambda i: (i, 0)),
            pl.BlockSpec((1, MEM_DIM), lambda i: (0, 0)),
            pl.BlockSpec((1, MEM_DIM), lambda i: (0, 0)),
            pl.BlockSpec((MEM_DIM, HO), lambda i: (0, 0)),
        ],
        out_specs=pl.BlockSpec((E_BR, 128, HO), lambda i: (i, 0, 0)),
        out_shape=jax.ShapeDtypeStruct((E_ROWS, 128, HO), jnp.float32),
    )(t2, w2, b2, Wt)


def _sc_edges(src, dst, etab, srctab, dsttab, zeros):
    mesh = plsc.VectorSubcoreMesh(core_axis_name="c", subcore_axis_name="s")

    H_CH = CHUNK // 2

    @functools.partial(
        pl.kernel,
        mesh=mesh,
        out_type=jax.ShapeDtypeStruct((NUM_CORES, N_PAD, ACC_W), jnp.float32),
        scratch_types=[
            pltpu.VMEM_SHARED((N_PAD, ACC_W), jnp.float32),
            pltpu.VMEM((2, 2, H_CH), jnp.int32),
            pltpu.VMEM((2, 2, H_CH), jnp.int32),
            pltpu.VMEM((CHUNK, 2 * HO), jnp.float32),
            pltpu.VMEM((CHUNK, 2 * HO), jnp.float32),
            pltpu.VMEM((CHUNK, HO), jnp.float32),
            pltpu.VMEM((CHUNK, ACC_W), jnp.float32),
            pltpu.SemaphoreType.DMA,
            pltpu.SemaphoreType.DMA,
            pltpu.SemaphoreType.DMA,
            pltpu.SemaphoreType.DMA,
            pltpu.SemaphoreType.DMA,
        ],
    )
    def k(src_hbm, dst_hbm, etab_hbm, srctab_hbm, dsttab_hbm, zeros_hbm,
          out_hbm, acc_sh, sidx_v, didx_v, srow_v, drow_v, erow_v, crow_v,
          sem_a, sem_b, sem_i, sem_sa, sem_sb):
        ci = lax.axis_index("c")
        si = lax.axis_index("s")
        wid = si * NUM_CORES + ci

        row0 = si * ROWS_PER_TILE
        pltpu.sync_copy(zeros_hbm.at[pl.ds(row0, ROWS_PER_TILE)],
                        acc_sh.at[pl.ds(row0, ROWS_PER_TILE)])
        plsc.subcore_barrier()

        lanes = lax.iota(jnp.int32, 16)
        perms = [lanes ^ d for d in (1, 2, 4, 8)]

        dnums = lax.GatherDimensionNumbers(
            offset_dims=(), collapsed_slice_dims=(0,), start_index_map=(0,))

        def lane_gather(v, p):
            return lax.gather(v, p[:, None], dnums, (1,),
                              mode=lax.GatherScatterMode.PROMISE_IN_BOUNDS)

        def allsum(v):
            for p in perms:
                v = v + lane_gather(v, p)
            return v

        def idx_copies(j, par):
            base = wid * EPW + j * CHUNK
            return [
                pltpu.make_async_copy(src_hbm.at[pl.ds(base, H_CH)],
                                      sidx_v.at[par, 0], sem_i),
                pltpu.make_async_copy(src_hbm.at[pl.ds(base + H_CH, H_CH)],
                                      sidx_v.at[par, 1], sem_i),
                pltpu.make_async_copy(dst_hbm.at[pl.ds(base, H_CH)],
                                      didx_v.at[par, 0], sem_i),
                pltpu.make_async_copy(dst_hbm.at[pl.ds(base + H_CH, H_CH)],
                                      didx_v.at[par, 1], sem_i),
            ]

        def half_gathers(j, par, h, sem):
            base = wid * EPW + j * CHUNK + h * H_CH
            off = h * H_CH
            return [
                pltpu.make_async_copy(srctab_hbm.at[sidx_v.at[par, h]],
                                      srow_v.at[pl.ds(off, H_CH)], sem),
                pltpu.make_async_copy(dsttab_hbm.at[didx_v.at[par, h]],
                                      drow_v.at[pl.ds(off, H_CH)], sem),
                pltpu.make_async_copy(etab_hbm.at[pl.ds(base, H_CH)],
                                      erow_v.at[pl.ds(off, H_CH)], sem),
            ]

        def half_scatter(par, h, sem):
            off = h * H_CH
            return pltpu.make_async_copy(crow_v.at[pl.ds(off, H_CH)],
                                         acc_sh.at[didx_v.at[par, h]], sem)

        def compute_half(h):
            off = h * H_CH

            @plsc.parallel_loop(0, H_CH, step=1, unroll=2)
            def edge_body(i):
                r = off + i
                e0 = erow_v[r, pl.ds(0, 16)]
                e1 = erow_v[r, pl.ds(16, 16)]
                e2 = erow_v[r, pl.ds(32, 16)]
                e3 = erow_v[r, pl.ds(48, 16)]
                q0 = drow_v[r, pl.ds(0, 16)]
                q1 = drow_v[r, pl.ds(16, 16)]
                q2 = drow_v[r, pl.ds(32, 16)]
                q3 = drow_v[r, pl.ds(48, 16)]
                k0 = srow_v[r, pl.ds(0, 16)] + e0
                k1 = srow_v[r, pl.ds(16, 16)] + e1
                k2 = srow_v[r, pl.ds(32, 16)] + e2
                k3 = srow_v[r, pl.ds(48, 16)] + e3
                w0 = jnp.exp(allsum(q0 * k0 + q1 * k1) * SCALE)
                w1 = jnp.exp(allsum(q2 * k2 + q3 * k3) * SCALE)
                v0 = srow_v[r, pl.ds(64, 16)] + e0
                v1 = srow_v[r, pl.ds(80, 16)] + e1
                v2 = srow_v[r, pl.ds(96, 16)] + e2
                v3 = srow_v[r, pl.ds(112, 16)] + e3
                crow_v[r, pl.ds(0, 16)] = w0 * v0
                crow_v[r, pl.ds(16, 16)] = w0 * v1
                crow_v[r, pl.ds(32, 16)] = w1 * v2
                crow_v[r, pl.ds(48, 16)] = w1 * v3
                crow_v[r, pl.ds(64, 16)] = jnp.where(
                    lanes == 0, w0, jnp.where(lanes == 1, w1, 0.0))

        def chunk_steps(j, par, first, last):
            if not first:
                half_scatter(1 - par, 0, sem_sa).wait()
                half_scatter(1 - par, 1, sem_sb).wait()
            if not last:
                for c in idx_copies(j + 1, 1 - par):
                    c.start()
            for c in half_gathers(j, par, 0, sem_a):
                c.wait()
            for c in half_gathers(j, par, 1, sem_b):
                c.start()
            compute_half(0)
            half_scatter(par, 0, sem_sa).start(add=True)
            if not last:
                for c in idx_copies(j + 1, 1 - par):
                    c.wait()
                for c in half_gathers(j + 1, 1 - par, 0, sem_a):
                    c.start()
            for c in half_gathers(j, par, 1, sem_b):
                c.wait()
            compute_half(1)
            half_scatter(par, 1, sem_sb).start(add=True)

        for c in idx_copies(0, 0):
            c.start()
        for c in idx_copies(0, 0):
            c.wait()
        for c in half_gathers(0, 0, 0, sem_a):
            c.start()

        def pair_body(jj, _):
            j = 2 * jj
            chunk_steps(j, 0, first=False, last=False)
            chunk_steps(j + 1, 1, first=False, last=False)
            return 0

        chunk_steps(0, 0, first=True, last=False)
        chunk_steps(1, 1, first=False, last=False)

        lax.fori_loop(1, (NCHUNK - 1) // 2, pair_body, 0)

        chunk_steps(NCHUNK - 1, 0, first=False, last=True)
        half_scatter(0, 0, sem_sa).wait()
        half_scatter(0, 1, sem_sb).wait()

        plsc.subcore_barrier()
        pltpu.sync_copy(acc_sh.at[pl.ds(row0, ROWS_PER_TILE)],
                        out_hbm.at[ci, pl.ds(row0, ROWS_PER_TILE)])

    return k(src, dst, etab, srctab, dsttab, zeros)


def _final_body(p_ref, out_ref):
    p = p_ref[...]
    acc = p[0] + p[1]
    num = acc[:, 0:HO]
    d0 = acc[:, HO:HO + 1]
    d1 = acc[:, HO + 1:HO + 2]
    out_ref[...] = jnp.concatenate(
        [num[:, 0:OUT_C] / (d0 + 1e-16), num[:, OUT_C:HO] / (d1 + 1e-16)],
        axis=1)


def _finalize(parts):
    return pl.pallas_call(
        _final_body,
        out_shape=jax.ShapeDtypeStruct((N_PAD, HO), jnp.float32),
    )(parts)


def kernel(src, dst, t, x, memory, w_time, b_time, Wq, Wk, Wv, Wt, Wm):
    src = src.astype(jnp.int32)
    dst = dst.astype(jnp.int32)
    dsttab, srctab = _node_tables(x, memory, Wq, Wk, Wv, Wm)
    t_pad = jnp.concatenate(
        [t, jnp.zeros((E_ROWS * 128 - N_EDGES,), jnp.float32)]
    ).reshape(E_ROWS, 128)
    etab = _etab(t_pad, w_time.reshape(1, MEM_DIM),
                 b_time.reshape(1, MEM_DIM), Wt).reshape(E_ROWS * 128, HO)
    zeros = jnp.zeros((N_PAD, ACC_W), jnp.float32)
    parts = _sc_edges(src, dst, etab, srctab, dsttab, zeros)
    return _finalize(parts)[:N_NODES]
